# baseline (speedup 1.0000x reference)
import functools

import jax
import jax.numpy as jnp
from jax import lax
from jax.experimental import pallas as pl
from jax.experimental.pallas import tpu as pltpu
from jax.experimental.pallas import tpu_sc as plsc

C = 4096
L = 16
NCHUNK = C // L
NC = 2
NS = 16
NW = NC * NS
RPW = C // NW
LABEL_SMOOTH = 0.1
EPS = LABEL_SMOOTH / (C - 1)
HI = 1.0 - LABEL_SMOOTH


GRP = 16
NGRP = 256 // GRP


def _sc_body(preds_hbm, tgt_hbm, stats_hbm, cval_hbm, cidx_hbm,
             ra0, ra1, ra2, ra3, rb0, rb1, rb2, rb3, tgtbuf,
             gb0, gb1, gb2, gb3, trv, tri,
             stats_v, cval_v, cidx_v, sema, semb):
    wid = lax.axis_index("s") * NC + lax.axis_index("c")
    base = wid * RPW
    pltpu.sync_copy(tgt_hbm.at[pl.ds(base, RPW)], tgtbuf)

    iota = lax.iota(jnp.int32, L)
    zi = jnp.zeros((L,), jnp.int32)
    ninf = jnp.full((L,), -jnp.inf, jnp.float32)

    pltpu.async_copy(preds_hbm.at[base], ra0, sema)
    pltpu.async_copy(preds_hbm.at[base + 1], ra1, sema)
    pltpu.async_copy(preds_hbm.at[base + 2], ra2, sema)
    pltpu.async_copy(preds_hbm.at[base + 3], ra3, sema)

    def do_quad(j, pbufs, psem, obufs, osem):
        row = base + j
        for pb in pbufs:
            pltpu.make_async_copy(preds_hbm.at[base], pb, psem).wait()

        @pl.when(j + 4 < RPW)
        def _():
            for q, ob in enumerate(obufs):
                pltpu.async_copy(preds_hbm.at[row + 4 + q], ob, osem)

        gbufs = (gb0, gb1, gb2, gb3)

        def ga(g, carry):
            m0, m1, m2, m3 = carry

            def gi(t, c2):
                a0, a1, a2, a3 = c2
                o = (g * GRP + t) * L
                return (jnp.maximum(a0, pbufs[0][pl.ds(o, L)]),
                        jnp.maximum(a1, pbufs[1][pl.ds(o, L)]),
                        jnp.maximum(a2, pbufs[2][pl.ds(o, L)]),
                        jnp.maximum(a3, pbufs[3][pl.ds(o, L)]))

            a0, a1, a2, a3 = lax.fori_loop(
                0, GRP, gi, (ninf, ninf, ninf, ninf), unroll=8)
            gb0[pl.ds(g * L, L)] = a0
            gb1[pl.ds(g * L, L)] = a1
            gb2[pl.ds(g * L, L)] = a2
            gb3[pl.ds(g * L, L)] = a3
            return (jnp.maximum(m0, a0), jnp.maximum(m1, a1),
                    jnp.maximum(m2, a2), jnp.maximum(m3, a3))

        m4 = lax.fori_loop(0, NGRP, ga, (ninf, ninf, ninf, ninf))

        bcast3 = jnp.full((L,), 3, jnp.int32)
        taus = []
        for q in range(4):
            srt, _ = plsc.sort_key_val(m4[q], m4[q], descending=True)
            taus.append(srt.at[bcast3].get(mode="promise_in_bounds"))

        for q in range(4):
            for off in (0, 16, 32, 48):
                trv[pl.ds(64 * q + off, L)] = ninf
                tri[pl.ds(64 * q + off, L)] = zi

        def gbq(g, _):
            for q in range(4):
                gm = gbufs[q][pl.ds(g * L, L)]
                hit = jnp.any(gm >= taus[q])
                rowbuf = pbufs[q]
                toff = 64 * q

                @pl.when(hit)
                def _(rowbuf=rowbuf, toff=toff):
                    r0v = trv[pl.ds(toff, L)]
                    r1v = trv[pl.ds(toff + 16, L)]
                    r2v = trv[pl.ds(toff + 32, L)]
                    r3v = trv[pl.ds(toff + 48, L)]
                    r0i = tri[pl.ds(toff, L)]
                    r1i = tri[pl.ds(toff + 16, L)]
                    r2i = tri[pl.ds(toff + 32, L)]
                    r3i = tri[pl.ds(toff + 48, L)]

                    def ins(k, carry):
                        c0v, c1v, c2v, c3v, c0i, c1i, c2i, c3i = carry
                        c = g * GRP + k
                        v = rowbuf[pl.ds(c * L, L)]
                        cols = iota + c * L
                        w0 = v > c0v
                        w1 = v > c1v
                        w2 = v > c2v
                        w3 = v > c3v
                        n0v = jnp.where(w0, v, c0v)
                        n0i = jnp.where(w0, cols, c0i)
                        n1v = jnp.where(w0, c0v, jnp.where(w1, v, c1v))
                        n1i = jnp.where(w0, c0i, jnp.where(w1, cols, c1i))
                        n2v = jnp.where(w1, c1v, jnp.where(w2, v, c2v))
                        n2i = jnp.where(w1, c1i, jnp.where(w2, cols, c2i))
                        n3v = jnp.where(w2, c2v, jnp.where(w3, v, c3v))
                        n3i = jnp.where(w2, c2i, jnp.where(w3, cols, c3i))
                        return (n0v, n1v, n2v, n3v, n0i, n1i, n2i, n3i)

                    r0v, r1v, r2v, r3v, r0i, r1i, r2i, r3i = lax.fori_loop(
                        0, GRP, ins,
                        (r0v, r1v, r2v, r3v, r0i, r1i, r2i, r3i), unroll=4)
                    trv[pl.ds(toff, L)] = r0v
                    trv[pl.ds(toff + 16, L)] = r1v
                    trv[pl.ds(toff + 32, L)] = r2v
                    trv[pl.ds(toff + 48, L)] = r3v
                    tri[pl.ds(toff, L)] = r0i
                    tri[pl.ds(toff + 16, L)] = r1i
                    tri[pl.ds(toff + 32, L)] = r2i
                    tri[pl.ds(toff + 48, L)] = r3i

            return 0

        lax.fori_loop(0, NGRP, gbq, 0)

        for q in range(4):
            tq = plsc.load_gather(tgtbuf, [jnp.full((L,), j + q, jnp.int32)])
            pq = plsc.load_gather(pbufs[q], [tq])
            stats_v[pl.ds((j + q) * L, L)] = pq
            sb = (j + q) * 64
            toff = 64 * q
            cval_v[pl.ds(sb, L)] = trv[pl.ds(toff, L)]
            cval_v[pl.ds(sb + 16, L)] = trv[pl.ds(toff + 16, L)]
            cval_v[pl.ds(sb + 32, L)] = trv[pl.ds(toff + 32, L)]
            cval_v[pl.ds(sb + 48, L)] = trv[pl.ds(toff + 48, L)]
            cidx_v[pl.ds(sb, L)] = tri[pl.ds(toff, L)]
            cidx_v[pl.ds(sb + 16, L)] = tri[pl.ds(toff + 16, L)]
            cidx_v[pl.ds(sb + 32, L)] = tri[pl.ds(toff + 32, L)]
            cidx_v[pl.ds(sb + 48, L)] = tri[pl.ds(toff + 48, L)]

    def oct_step(t, _):
        do_quad(8 * t, (ra0, ra1, ra2, ra3), sema, (rb0, rb1, rb2, rb3), semb)
        do_quad(8 * t + 4, (rb0, rb1, rb2, rb3), semb,
                (ra0, ra1, ra2, ra3), sema)
        return 0

    lax.fori_loop(0, RPW // 8, oct_step, 0)

    pltpu.sync_copy(stats_v, stats_hbm.at[pl.ds(base * L, RPW * L)])
    pltpu.sync_copy(cval_v, cval_hbm.at[pl.ds(base * 64, RPW * 64)])
    pltpu.sync_copy(cidx_v, cidx_hbm.at[pl.ds(base * 64, RPW * 64)])


def _fin_body(preds_ref, pt_ref, cval_ref, cidx_ref, tgt_ref, out_ref):
    i = pl.program_id(0)
    R = pt_ref.shape[0]
    x = preds_ref[...]
    M = jnp.max(x, axis=1)
    S = jnp.sum(jnp.exp(x - M[:, None]), axis=1)
    rowsum = jnp.sum(x, axis=1)
    pt = jnp.max(pt_ref[...], axis=1)

    lse = M + jnp.log(S)
    full = EPS * (rowsum - C * lse) + (HI - EPS) * (pt - lse)

    cval = cval_ref[...]
    cidx = cidx_ref[...]
    alive = jnp.ones(cval.shape, jnp.bool_)
    tv = []
    ti = []
    for _ in range(4):
        mv = jnp.where(alive, cval, -jnp.inf)
        cur = jnp.max(mv, axis=1)
        cand = mv == cur[:, None]
        curi = jnp.min(jnp.where(cand, cidx, C), axis=1)
        tv.append(cur)
        ti.append(curi)
        alive = alive & ~(cand & (cidx == curi[:, None]))

    rows = i * R + lax.broadcasted_iota(jnp.int32, (R,), 0)
    in0 = ti[0] == rows
    in1 = ti[1] == rows
    in2 = ti[2] == rows
    tgt = tgt_ref[:, 0]

    def term(v, idx):
        w = jnp.where(idx == tgt, HI, EPS)
        return w * (v - lse)

    sk0 = jnp.where(in0, term(tv[1], ti[1]), term(tv[0], ti[0]))
    sk1 = jnp.where(in0 | in1, term(tv[2], ti[2]), term(tv[1], ti[1]))
    sk2 = jnp.where(in0 | in1 | in2, term(tv[3], ti[3]), term(tv[2], ti[2]))
    skipped = sk0 + sk1 + sk2

    loss = -(full - skipped)
    part = jnp.reshape(jnp.sum(loss) * (1.0 / C), (1, 1))

    @pl.when(i == 0)
    def _():
        out_ref[...] = jnp.zeros((1, 1), jnp.float32)

    out_ref[...] += part


def _sc_call(preds2d, targets):
    mesh = plsc.VectorSubcoreMesh(core_axis_name="c", subcore_axis_name="s",
                                  num_cores=NC, num_subcores=NS)
    f = functools.partial(
        pl.kernel,
        mesh=mesh,
        out_type=[
            jax.ShapeDtypeStruct((C * L,), jnp.float32),
            jax.ShapeDtypeStruct((C * 64,), jnp.float32),
            jax.ShapeDtypeStruct((C * 64,), jnp.int32),
        ],
        scratch_types=(
            [pltpu.VMEM((C,), jnp.float32)] * 8 + [
            pltpu.VMEM((RPW,), jnp.int32)] +
            [pltpu.VMEM((NGRP * L,), jnp.float32)] * 4 + [
            pltpu.VMEM((256,), jnp.float32),
            pltpu.VMEM((256,), jnp.int32),
            pltpu.VMEM((RPW * L,), jnp.float32),
            pltpu.VMEM((RPW * 64,), jnp.float32),
            pltpu.VMEM((RPW * 64,), jnp.int32),
            pltpu.SemaphoreType.DMA,
            pltpu.SemaphoreType.DMA,
        ]),
        compiler_params=pltpu.CompilerParams(needs_layout_passes=False),
    )(_sc_body)
    return f(preds2d, targets)


def kernel(preds, targets):
    tgt = targets.astype(jnp.int32)
    ptv, cval, cidx = _sc_call(preds, tgt)

    R = 512
    out = pl.pallas_call(
        _fin_body,
        grid=(C // R,),
        in_specs=[
            pl.BlockSpec((R, C), lambda i: (i, 0)),
            pl.BlockSpec((R, 16), lambda i: (i, 0)),
            pl.BlockSpec((R, 64), lambda i: (i, 0)),
            pl.BlockSpec((R, 64), lambda i: (i, 0)),
            pl.BlockSpec((R, 1), lambda i: (i, 0)),
        ],
        out_specs=pl.BlockSpec((1, 1), lambda i: (0, 0)),
        out_shape=jax.ShapeDtypeStruct((1, 1), jnp.float32),
    )(preds, ptv.reshape(C, 16), cval.reshape(C, 64), cidx.reshape(C, 64),
      tgt.reshape(C, 1))
    return out[0, 0]


# speedup vs baseline: 1.0999x; 1.0999x over previous
import functools

import jax
import jax.numpy as jnp
from jax import lax
from jax.experimental import pallas as pl
from jax.experimental.pallas import tpu as pltpu
from jax.experimental.pallas import tpu_sc as plsc

C = 4096
L = 16
NCHUNK = C // L
NC = 2
NS = 16
NW = NC * NS
RPW = C // NW
LABEL_SMOOTH = 0.1
EPS = LABEL_SMOOTH / (C - 1)
HI = 1.0 - LABEL_SMOOTH


GRP = 16
NGRP = 256 // GRP


def _sc_body(preds_hbm, tgt_hbm, stats_hbm, cval_hbm, cidx_hbm,
             ra0, ra1, ra2, ra3, rb0, rb1, rb2, rb3, tgtbuf,
             gb0, gb1, gb2, gb3, trv, tri,
             stats_v, cval_v, cidx_v, sema, semb):
    wid = lax.axis_index("s") * NC + lax.axis_index("c")
    base = wid * RPW
    pltpu.sync_copy(tgt_hbm.at[pl.ds(base, RPW)], tgtbuf)

    iota = lax.iota(jnp.int32, L)
    zi = jnp.zeros((L,), jnp.int32)
    ninf = jnp.full((L,), -jnp.inf, jnp.float32)

    pltpu.async_copy(preds_hbm.at[base], ra0, sema)
    pltpu.async_copy(preds_hbm.at[base + 1], ra1, sema)
    pltpu.async_copy(preds_hbm.at[base + 2], ra2, sema)
    pltpu.async_copy(preds_hbm.at[base + 3], ra3, sema)

    def do_quad(j, pbufs, psem, obufs, osem):
        row = base + j
        for pb in pbufs:
            pltpu.make_async_copy(preds_hbm.at[base], pb, psem).wait()

        @pl.when(j + 4 < RPW)
        def _():
            for q, ob in enumerate(obufs):
                pltpu.async_copy(preds_hbm.at[row + 4 + q], ob, osem)

        gbufs = (gb0, gb1, gb2, gb3)

        def ga(g, carry):
            m0, m1, m2, m3 = carry

            def gi(t, c2):
                a0, a1, a2, a3 = c2
                o = (g * GRP + t) * L
                return (jnp.maximum(a0, pbufs[0][pl.ds(o, L)]),
                        jnp.maximum(a1, pbufs[1][pl.ds(o, L)]),
                        jnp.maximum(a2, pbufs[2][pl.ds(o, L)]),
                        jnp.maximum(a3, pbufs[3][pl.ds(o, L)]))

            a0, a1, a2, a3 = lax.fori_loop(
                0, GRP, gi, (ninf, ninf, ninf, ninf), unroll=8)
            gb0[pl.ds(g * L, L)] = a0
            gb1[pl.ds(g * L, L)] = a1
            gb2[pl.ds(g * L, L)] = a2
            gb3[pl.ds(g * L, L)] = a3
            return (jnp.maximum(m0, a0), jnp.maximum(m1, a1),
                    jnp.maximum(m2, a2), jnp.maximum(m3, a3))

        m4 = lax.fori_loop(0, NGRP, ga, (ninf, ninf, ninf, ninf))

        bcast3 = jnp.full((L,), 3, jnp.int32)
        taus = []
        for q in range(4):
            srt, _ = plsc.sort_key_val(m4[q], m4[q], descending=True)
            taus.append(srt.at[bcast3].get(mode="promise_in_bounds"))

        for q in range(4):
            for off in (0, 16, 32, 48):
                trv[pl.ds(64 * q + off, L)] = ninf
                tri[pl.ds(64 * q + off, L)] = zi

        def gbq(g, _):
            for q in range(4):
                gm = gbufs[q][pl.ds(g * L, L)]
                hit = jnp.any(gm >= taus[q])
                rowbuf = pbufs[q]
                toff = 64 * q

                @pl.when(hit)
                def _(rowbuf=rowbuf, toff=toff):
                    r0v = trv[pl.ds(toff, L)]
                    r1v = trv[pl.ds(toff + 16, L)]
                    r2v = trv[pl.ds(toff + 32, L)]
                    r3v = trv[pl.ds(toff + 48, L)]
                    r0i = tri[pl.ds(toff, L)]
                    r1i = tri[pl.ds(toff + 16, L)]
                    r2i = tri[pl.ds(toff + 32, L)]
                    r3i = tri[pl.ds(toff + 48, L)]

                    def ins(k, carry):
                        c0v, c1v, c2v, c3v, c0i, c1i, c2i, c3i = carry
                        c = g * GRP + k
                        v = rowbuf[pl.ds(c * L, L)]
                        cols = iota + c * L
                        w0 = v > c0v
                        w1 = v > c1v
                        w2 = v > c2v
                        w3 = v > c3v
                        n0v = jnp.where(w0, v, c0v)
                        n0i = jnp.where(w0, cols, c0i)
                        n1v = jnp.where(w0, c0v, jnp.where(w1, v, c1v))
                        n1i = jnp.where(w0, c0i, jnp.where(w1, cols, c1i))
                        n2v = jnp.where(w1, c1v, jnp.where(w2, v, c2v))
                        n2i = jnp.where(w1, c1i, jnp.where(w2, cols, c2i))
                        n3v = jnp.where(w2, c2v, jnp.where(w3, v, c3v))
                        n3i = jnp.where(w2, c2i, jnp.where(w3, cols, c3i))
                        return (n0v, n1v, n2v, n3v, n0i, n1i, n2i, n3i)

                    r0v, r1v, r2v, r3v, r0i, r1i, r2i, r3i = lax.fori_loop(
                        0, GRP, ins,
                        (r0v, r1v, r2v, r3v, r0i, r1i, r2i, r3i), unroll=4)
                    trv[pl.ds(toff, L)] = r0v
                    trv[pl.ds(toff + 16, L)] = r1v
                    trv[pl.ds(toff + 32, L)] = r2v
                    trv[pl.ds(toff + 48, L)] = r3v
                    tri[pl.ds(toff, L)] = r0i
                    tri[pl.ds(toff + 16, L)] = r1i
                    tri[pl.ds(toff + 32, L)] = r2i
                    tri[pl.ds(toff + 48, L)] = r3i

            return 0

        lax.fori_loop(0, NGRP, gbq, 0)

        for q in range(4):
            tq = plsc.load_gather(tgtbuf, [jnp.full((L,), j + q, jnp.int32)])
            pq = plsc.load_gather(pbufs[q], [tq])
            stats_v[pl.ds((j + q) * L, L)] = pq
            sb = (j + q) * 64
            toff = 64 * q
            cval_v[pl.ds(sb, L)] = trv[pl.ds(toff, L)]
            cval_v[pl.ds(sb + 16, L)] = trv[pl.ds(toff + 16, L)]
            cval_v[pl.ds(sb + 32, L)] = trv[pl.ds(toff + 32, L)]
            cval_v[pl.ds(sb + 48, L)] = trv[pl.ds(toff + 48, L)]
            cidx_v[pl.ds(sb, L)] = tri[pl.ds(toff, L)]
            cidx_v[pl.ds(sb + 16, L)] = tri[pl.ds(toff + 16, L)]
            cidx_v[pl.ds(sb + 32, L)] = tri[pl.ds(toff + 32, L)]
            cidx_v[pl.ds(sb + 48, L)] = tri[pl.ds(toff + 48, L)]

    def oct_step(t, _):
        do_quad(8 * t, (ra0, ra1, ra2, ra3), sema, (rb0, rb1, rb2, rb3), semb)
        do_quad(8 * t + 4, (rb0, rb1, rb2, rb3), semb,
                (ra0, ra1, ra2, ra3), sema)
        return 0

    lax.fori_loop(0, RPW // 8, oct_step, 0)

    pltpu.sync_copy(stats_v, stats_hbm.at[pl.ds(base * L, RPW * L)])
    pltpu.sync_copy(cval_v, cval_hbm.at[pl.ds(base * 64, RPW * 64)])
    pltpu.sync_copy(cidx_v, cidx_hbm.at[pl.ds(base * 64, RPW * 64)])


def _dense_body(preds_ref, out_ref):
    x = preds_ref[...]
    M = jnp.max(x, axis=1)
    S = jnp.sum(jnp.exp(x - M[:, None]), axis=1)
    rs = jnp.sum(x, axis=1)
    z = jnp.zeros_like(M)
    out_ref[...] = jnp.stack([M, S, rs, z, z, z, z, z], axis=0)


def _fin_body(dstats_ref, pt_ref, cval_ref, cidx_ref, tgt_ref, out_ref):
    i = pl.program_id(0)
    R = pt_ref.shape[0]
    dstats = dstats_ref[...]
    M = dstats[0, :]
    S = dstats[1, :]
    rowsum = dstats[2, :]
    pt = jnp.max(pt_ref[...], axis=1)

    lse = M + jnp.log(S)
    full = EPS * (rowsum - C * lse) + (HI - EPS) * (pt - lse)

    cval = cval_ref[...]
    cidx = cidx_ref[...]
    alive = jnp.ones(cval.shape, jnp.bool_)
    tv = []
    ti = []
    for _ in range(4):
        mv = jnp.where(alive, cval, -jnp.inf)
        cur = jnp.max(mv, axis=1)
        cand = mv == cur[:, None]
        curi = jnp.min(jnp.where(cand, cidx, C), axis=1)
        tv.append(cur)
        ti.append(curi)
        alive = alive & ~(cand & (cidx == curi[:, None]))

    rows = i * R + lax.broadcasted_iota(jnp.int32, (R,), 0)
    in0 = ti[0] == rows
    in1 = ti[1] == rows
    in2 = ti[2] == rows
    tgt = tgt_ref[:, 0]

    def term(v, idx):
        w = jnp.where(idx == tgt, HI, EPS)
        return w * (v - lse)

    sk0 = jnp.where(in0, term(tv[1], ti[1]), term(tv[0], ti[0]))
    sk1 = jnp.where(in0 | in1, term(tv[2], ti[2]), term(tv[1], ti[1]))
    sk2 = jnp.where(in0 | in1 | in2, term(tv[3], ti[3]), term(tv[2], ti[2]))
    skipped = sk0 + sk1 + sk2

    loss = -(full - skipped)
    part = jnp.reshape(jnp.sum(loss) * (1.0 / C), (1, 1))

    @pl.when(i == 0)
    def _():
        out_ref[...] = jnp.zeros((1, 1), jnp.float32)

    out_ref[...] += part


def _sc_call(preds2d, targets):
    mesh = plsc.VectorSubcoreMesh(core_axis_name="c", subcore_axis_name="s",
                                  num_cores=NC, num_subcores=NS)
    f = functools.partial(
        pl.kernel,
        mesh=mesh,
        out_type=[
            jax.ShapeDtypeStruct((C * L,), jnp.float32),
            jax.ShapeDtypeStruct((C * 64,), jnp.float32),
            jax.ShapeDtypeStruct((C * 64,), jnp.int32),
        ],
        scratch_types=(
            [pltpu.VMEM((C,), jnp.float32)] * 8 + [
            pltpu.VMEM((RPW,), jnp.int32)] +
            [pltpu.VMEM((NGRP * L,), jnp.float32)] * 4 + [
            pltpu.VMEM((256,), jnp.float32),
            pltpu.VMEM((256,), jnp.int32),
            pltpu.VMEM((RPW * L,), jnp.float32),
            pltpu.VMEM((RPW * 64,), jnp.float32),
            pltpu.VMEM((RPW * 64,), jnp.int32),
            pltpu.SemaphoreType.DMA,
            pltpu.SemaphoreType.DMA,
        ]),
        compiler_params=pltpu.CompilerParams(needs_layout_passes=False),
    )(_sc_body)
    return f(preds2d, targets)


def kernel(preds, targets):
    tgt = targets.astype(jnp.int32)
    ptv, cval, cidx = _sc_call(preds, tgt)

    RD = 256
    dstats = pl.pallas_call(
        _dense_body,
        grid=(C // RD,),
        in_specs=[pl.BlockSpec((RD, C), lambda i: (i, 0))],
        out_specs=pl.BlockSpec((8, RD), lambda i: (0, i)),
        out_shape=jax.ShapeDtypeStruct((8, C), jnp.float32),
    )(preds)

    R = 512
    out = pl.pallas_call(
        _fin_body,
        grid=(C // R,),
        in_specs=[
            pl.BlockSpec((8, R), lambda i: (0, i)),
            pl.BlockSpec((R, 16), lambda i: (i, 0)),
            pl.BlockSpec((R, 64), lambda i: (i, 0)),
            pl.BlockSpec((R, 64), lambda i: (i, 0)),
            pl.BlockSpec((R, 1), lambda i: (i, 0)),
        ],
        out_specs=pl.BlockSpec((1, 1), lambda i: (0, 0)),
        out_shape=jax.ShapeDtypeStruct((1, 1), jnp.float32),
    )(dstats, ptv.reshape(C, 16), cval.reshape(C, 64), cidx.reshape(C, 64),
      tgt.reshape(C, 1))
    return out[0, 0]
